# baseline (speedup 1.0000x reference)
import functools

import jax
import jax.numpy as jnp
from jax import lax
from jax.experimental import pallas as pl
from jax.experimental.pallas import tpu as pltpu
from jax.experimental.pallas import tpu_sc as plsc

NC = 2
NS = 16
NW = NC * NS


@functools.lru_cache(maxsize=None)
def _build(n, d):
    per_w = n // NW
    chunk = 32
    nbuf = 8
    n_chunks = per_w // chunk
    n_groups = n_chunks // nbuf
    mesh = plsc.VectorSubcoreMesh(core_axis_name="c", subcore_axis_name="s")

    @functools.partial(
        pl.kernel,
        out_type=jax.ShapeDtypeStruct((n, d), jnp.float32),
        mesh=mesh,
        scratch_types=[
            pltpu.VMEM((per_w,), jnp.int32),
            [pltpu.VMEM((chunk, d), jnp.float32) for _ in range(nbuf)],
            [pltpu.SemaphoreType.DMA for _ in range(nbuf)],
            [pltpu.SemaphoreType.DMA for _ in range(nbuf)],
        ],
    )
    def gather_kernel(idx_hbm, table_hbm, out_hbm, idx_v, bufs, gsems, wsems):
        wid = lax.axis_index("s") * NC + lax.axis_index("c")
        base = wid * per_w
        pltpu.sync_copy(idx_hbm.at[pl.ds(base, per_w)], idx_v)

        def fire_gather(i, b):
            pltpu.async_copy(
                table_hbm.at[idx_v.at[pl.ds(i * chunk, chunk)]],
                bufs[b], gsems[b],
            )

        def fire_write(i, b):
            pltpu.async_copy(
                bufs[b], out_hbm.at[pl.ds(base + i * chunk, chunk)], wsems[b]
            )

        def drain(sem, buf):
            pltpu.make_async_copy(out_hbm.at[pl.ds(0, chunk)], buf, sem).wait()

        for b in range(nbuf):
            fire_gather(b, b)

        def group_body(g, _):
            i0 = g * nbuf
            for b in range(nbuf):
                drain(gsems[b], bufs[b])
                fire_write(i0 + b, b)
                drain(wsems[b], bufs[b])
                fire_gather(i0 + nbuf + b, b)
            return 0

        lax.fori_loop(0, n_groups - 1, group_body, 0)

        i0 = (n_groups - 1) * nbuf
        for b in range(nbuf):
            drain(gsems[b], bufs[b])
            fire_write(i0 + b, b)
            drain(wsems[b], bufs[b])

    return gather_kernel


def kernel(indices, table):
    b, l = indices.shape
    v, d = table.shape
    idx_t = jnp.transpose(indices.astype(jnp.int32)).reshape(-1)
    out = _build(b * l, d)(idx_t, table)
    return jnp.transpose(out.reshape(l, b, d), (1, 0, 2))


# speedup vs baseline: 1.0335x; 1.0335x over previous
import functools

import jax
import jax.numpy as jnp
from jax import lax
from jax.experimental import pallas as pl
from jax.experimental.pallas import tpu as pltpu
from jax.experimental.pallas import tpu_sc as plsc

NC = 2
NS = 16
NW = NC * NS


@functools.lru_cache(maxsize=None)
def _build(n, d):
    per_w = n // NW
    chunk = 64
    nbuf = 8
    n_chunks = per_w // chunk
    n_groups = n_chunks // nbuf
    mesh = plsc.VectorSubcoreMesh(core_axis_name="c", subcore_axis_name="s")

    @functools.partial(
        pl.kernel,
        out_type=jax.ShapeDtypeStruct((n, d), jnp.float32),
        mesh=mesh,
        scratch_types=[
            pltpu.VMEM((per_w,), jnp.int32),
            [pltpu.VMEM((chunk, d), jnp.float32) for _ in range(nbuf)],
            [pltpu.SemaphoreType.DMA for _ in range(nbuf)],
            [pltpu.SemaphoreType.DMA for _ in range(nbuf)],
        ],
    )
    def gather_kernel(idx_hbm, table_hbm, out_hbm, idx_v, bufs, gsems, wsems):
        wid = lax.axis_index("s") * NC + lax.axis_index("c")
        base = wid * per_w
        pltpu.sync_copy(idx_hbm.at[pl.ds(base, per_w)], idx_v)

        def fire_gather(i, b):
            pltpu.async_copy(
                table_hbm.at[idx_v.at[pl.ds(i * chunk, chunk)]],
                bufs[b], gsems[b],
            )

        def fire_write(i, b):
            pltpu.async_copy(
                bufs[b], out_hbm.at[pl.ds(base + i * chunk, chunk)], wsems[b]
            )

        def drain(sem, buf):
            pltpu.make_async_copy(out_hbm.at[pl.ds(0, chunk)], buf, sem).wait()

        for b in range(nbuf):
            fire_gather(b, b)

        def group_body(g, _):
            i0 = g * nbuf
            for b in range(nbuf):
                drain(gsems[b], bufs[b])
                fire_write(i0 + b, b)
                drain(wsems[b], bufs[b])
                fire_gather(i0 + nbuf + b, b)
            return 0

        lax.fori_loop(0, n_groups - 1, group_body, 0)

        i0 = (n_groups - 1) * nbuf
        for b in range(nbuf):
            drain(gsems[b], bufs[b])
            fire_write(i0 + b, b)
            drain(wsems[b], bufs[b])

    return gather_kernel


def kernel(indices, table):
    b, l = indices.shape
    v, d = table.shape
    idx_t = jnp.transpose(indices.astype(jnp.int32)).reshape(-1)
    out = _build(b * l, d)(idx_t, table)
    return jnp.transpose(out.reshape(l, b, d), (1, 0, 2))
